# baseline (speedup 1.0000x reference)
import functools

import jax
import jax.numpy as jnp
from jax import lax
from jax.experimental import pallas as pl
from jax.experimental.pallas import tpu as pltpu, tpu_sc as plsc

N = 10000
E = 320000
D = 128

NC = 2
NS = 16
NW = NC * NS

BLK = 125
CH = 20
NCH = 4
NP = 10240
RPT = NP // NS
ZB = 64
DW = 128

_mesh = plsc.VectorSubcoreMesh(core_axis_name="c", subcore_axis_name="s")


@functools.partial(
    pl.kernel,
    out_type=jax.ShapeDtypeStruct((NC, NP, DW), jnp.float32),
    mesh=_mesh,
    scratch_types=[
        pltpu.VMEM((CH, BLK), jnp.int32),
        pltpu.VMEM((BLK, DW), jnp.float32),
        pltpu.VMEM((BLK, DW), jnp.float32),
        pltpu.VMEM_SHARED((NP, DW), jnp.float32),
        pltpu.SemaphoreType.DMA,
    ],
)
def _sc_degree(dst_hbm, out_hbm, dstv, onesb, zb, acc, sem):
    c = lax.axis_index("c")
    s = lax.axis_index("s")
    wid = c * NS + s

    ones16 = jnp.full((16,), 1.0, jnp.float32)
    zero16 = jnp.zeros((16,), jnp.float32)

    def fill(i, carry):
        r = i // (DW // 16)
        t = i - r * (DW // 16)
        onesb[r, pl.ds(t * 16, 16)] = ones16
        zb[r, pl.ds(t * 16, 16)] = zero16
        return carry

    lax.fori_loop(0, BLK * (DW // 16), fill, 0)

    def zero_acc(i, carry):
        pltpu.sync_copy(zb.at[pl.ds(0, ZB)],
                        acc.at[pl.ds(s * RPT + i * ZB, ZB)])
        return carry

    lax.fori_loop(0, RPT // ZB, zero_acc, 0)
    plsc.subcore_barrier()

    def chunk(ch, carry):
        pltpu.sync_copy(dst_hbm.at[wid, ch], dstv)

        def blk(j, carry2):
            pltpu.async_copy(onesb, acc.at[dstv.at[j]], sem, add=True)
            return carry2

        lax.fori_loop(0, CH, blk, carry)

        def drain(j, carry2):
            pltpu.make_async_copy(onesb, acc.at[dstv.at[j]], sem).wait()
            return carry2

        return lax.fori_loop(0, CH, drain, carry)

    lax.fori_loop(0, NCH, chunk, 0)
    plsc.subcore_barrier()

    pltpu.sync_copy(acc.at[pl.ds(s * RPT, RPT)], out_hbm.at[c, pl.ds(s * RPT, RPT)])


@functools.partial(
    pl.kernel,
    out_type=jax.ShapeDtypeStruct((NC, NP, D), jnp.float32),
    mesh=_mesh,
    scratch_types=[
        pltpu.VMEM((CH, BLK), jnp.int32),
        pltpu.VMEM((CH, BLK), jnp.int32),
        pltpu.VMEM((BLK, D), jnp.float32),
        pltpu.VMEM((BLK, D), jnp.float32),
        pltpu.VMEM_SHARED((NP, D), jnp.float32),
        pltpu.SemaphoreType.DMA,
        pltpu.SemaphoreType.DMA,
    ],
)
def _sc_scatter(xs_hbm, src_hbm, dst_hbm, out_hbm, srcv, dstv, rows0, rows1,
                acc, sem0, sem1):
    c = lax.axis_index("c")
    s = lax.axis_index("s")
    wid = c * NS + s

    zero16 = jnp.zeros((16,), jnp.float32)

    def fill_zero(i, carry):
        r = i // (D // 16)
        t = i - r * (D // 16)
        rows0[r, pl.ds(t * 16, 16)] = zero16
        return carry

    lax.fori_loop(0, BLK * (D // 16), fill_zero, 0)

    def zero_acc(i, carry):
        pltpu.sync_copy(rows0.at[pl.ds(0, ZB)],
                        acc.at[pl.ds(s * RPT + i * ZB, ZB)])
        return carry

    lax.fori_loop(0, RPT // ZB, zero_acc, 0)
    plsc.subcore_barrier()

    def chunk(ch, carry):
        pltpu.sync_copy(src_hbm.at[wid, ch], srcv)
        pltpu.sync_copy(dst_hbm.at[wid, ch], dstv)
        pltpu.async_copy(xs_hbm.at[srcv.at[0]], rows0, sem0)

        def pair(i, carry2):
            j0 = 2 * i
            pltpu.async_copy(xs_hbm.at[srcv.at[j0 + 1]], rows1, sem1)
            pltpu.make_async_copy(xs_hbm.at[srcv.at[j0]], rows0, sem0).wait()
            pltpu.sync_copy(rows0, acc.at[dstv.at[j0]], add=True)
            pltpu.async_copy(xs_hbm.at[srcv.at[j0 + 2]], rows0, sem0)
            pltpu.make_async_copy(xs_hbm.at[srcv.at[j0 + 1]], rows1, sem1).wait()
            pltpu.sync_copy(rows1, acc.at[dstv.at[j0 + 1]], add=True)
            return carry2

        lax.fori_loop(0, CH // 2 - 1, pair, carry)
        pltpu.async_copy(xs_hbm.at[srcv.at[CH - 1]], rows1, sem1)
        pltpu.make_async_copy(xs_hbm.at[srcv.at[CH - 2]], rows0, sem0).wait()
        pltpu.sync_copy(rows0, acc.at[dstv.at[CH - 2]], add=True)
        pltpu.make_async_copy(xs_hbm.at[srcv.at[CH - 1]], rows1, sem1).wait()
        pltpu.sync_copy(rows1, acc.at[dstv.at[CH - 1]], add=True)
        return carry

    lax.fori_loop(0, NCH, chunk, 0)
    plsc.subcore_barrier()

    pltpu.sync_copy(acc.at[pl.ds(s * RPT, RPT)], out_hbm.at[c, pl.ds(s * RPT, RPT)])


def _mm(x, w):
    return lax.dot_general(x, w, (((1,), (1,)), ((), ())),
                           preferred_element_type=jnp.float32)


def _tc_mm1_body(x_ref, w1_ref, xw_ref):
    xw_ref[...] = _mm(x_ref[...], w1_ref[...])


def _tc_mm1(x, w1):
    return pl.pallas_call(
        _tc_mm1_body,
        out_shape=jax.ShapeDtypeStruct((N, D), jnp.float32),
    )(x, w1)


def _tc_prep_body(xw_ref, parts_ref, xs1_ref, dis_ref):
    deg = parts_ref[0, 0:N, 0:1] + parts_ref[1, 0:N, 0:1] + 1.0
    dis = lax.rsqrt(deg)
    dis_ref[...] = dis
    xs1_ref[...] = xw_ref[...] * dis


def _tc_prep(xw, parts):
    return pl.pallas_call(
        _tc_prep_body,
        out_shape=(
            jax.ShapeDtypeStruct((N, D), jnp.float32),
            jax.ShapeDtypeStruct((N, 1), jnp.float32),
        ),
    )(xw, parts)


def _bn_prelu(t, g, be, av):
    m = jnp.mean(t, axis=0, keepdims=True)
    v = jnp.mean((t - m) * (t - m), axis=0, keepdims=True)
    tn = (t - m) * lax.rsqrt(v + 1e-5) * g + be
    return jnp.where(tn >= 0, tn, av * tn)


def _tc_mid_body(acc_ref, xs1_ref, dis_ref, b1_ref, g1_ref, be1_ref, w2_ref,
                 a_ref, xs2_ref):
    dis = dis_ref[...]
    t = (acc_ref[0, 0:N] + acc_ref[1, 0:N] + xs1_ref[...]) * dis + b1_ref[...]
    h = _bn_prelu(t, g1_ref[...], be1_ref[...], a_ref[0, 0])
    xs2_ref[...] = _mm(h, w2_ref[...]) * dis


def _tc_mid(acc, xs1, dis, b1, g1, be1, w2, a):
    return pl.pallas_call(
        _tc_mid_body,
        out_shape=jax.ShapeDtypeStruct((N, D), jnp.float32),
    )(acc, xs1, dis, b1, g1, be1, w2, a)


def _tc_final_body(acc_ref, xs2_ref, dis_ref, b2_ref, g2_ref, be2_ref, a_ref,
                   out_ref):
    t = (acc_ref[0, 0:N] + acc_ref[1, 0:N] + xs2_ref[...]) * dis_ref[...] + b2_ref[...]
    out_ref[...] = _bn_prelu(t, g2_ref[...], be2_ref[...], a_ref[0, 0])


def _tc_final(acc, xs2, dis, b2, g2, be2, a):
    return pl.pallas_call(
        _tc_final_body,
        out_shape=jax.ShapeDtypeStruct((N, D), jnp.float32),
    )(acc, xs2, dis, b2, g2, be2, a)


def kernel(x, edge_index, W1, b1, g1, be1, W2, b2, g2, be2, a):
    ei = edge_index.astype(jnp.int32)
    src = ei[0].reshape(NW, NCH, CH, BLK)
    dst = ei[1].reshape(NW, NCH, CH, BLK)

    b1r = b1.reshape(1, D)
    g1r = g1.reshape(1, D)
    be1r = be1.reshape(1, D)
    b2r = b2.reshape(1, D)
    g2r = g2.reshape(1, D)
    be2r = be2.reshape(1, D)
    ar = jnp.asarray(a, jnp.float32).reshape(1, 1)

    deg_parts = _sc_degree(dst)
    xw1 = _tc_mm1(x, W1)
    xs1, dis = _tc_prep(xw1, deg_parts)
    acc1 = _sc_scatter(xs1, src, dst)
    xs2 = _tc_mid(acc1, xs1, dis, b1r, g1r, be1r, W2, ar)
    acc2 = _sc_scatter(xs2, src, dst)
    return _tc_final(acc2, xs2, dis, b2r, g2r, be2r, ar)
